# Initial kernel scaffold; baseline (speedup 1.0000x reference)
import jax
import jax.numpy as jnp
from jax.experimental import pallas as pl


def kernel(x, palette):
    raise NotImplementedError("write your pallas kernel here")


# speedup vs baseline: 2.6067x; 2.6067x over previous
import jax
import jax.numpy as jnp
from jax.experimental import pallas as pl
from jax.experimental.pallas import tpu as pltpu

_TILE = 1792


def _vq_body(x_ref, pal_ref, out_ref):
    xv = x_ref[0]
    T = xv.shape[1]
    pal = pal_ref[...]
    psq = jnp.sum(pal * pal, axis=1, keepdims=True)
    e = jax.lax.dot_general(
        pal.astype(jnp.bfloat16), xv.astype(jnp.bfloat16),
        (((1,), (0,)), ((), ())),
        preferred_element_type=jnp.float32)
    xsq = jnp.sum(xv * xv, axis=0, keepdims=True)
    scores = jnp.maximum((xsq + psq) - 2.0 * e, 0.0)
    K = pal.shape[0]
    iota = jax.lax.broadcasted_iota(jnp.int32, (K, T), 0)
    mval = jnp.min(scores, axis=0, keepdims=True)
    labels = jnp.min(jnp.where(scores == mval, iota, K), axis=0)
    onehot = (iota == labels[None, :]).astype(jnp.float32)
    rgb = jax.lax.dot_general(
        pal, onehot, (((0,), (0,)), ((), ())),
        precision=jax.lax.Precision.HIGHEST,
        preferred_element_type=jnp.float32)
    out_ref[0] = rgb


def kernel(x, palette):
    B, C, H, W = x.shape
    HW = H * W
    xr = x.reshape(B, C, HW)
    grid = (B, HW // _TILE)
    out = pl.pallas_call(
        _vq_body,
        grid=grid,
        in_specs=[
            pl.BlockSpec((1, C, _TILE), lambda b, i: (b, 0, i)),
            pl.BlockSpec(palette.shape, lambda b, i: (0, 0)),
        ],
        out_specs=pl.BlockSpec((1, C, _TILE), lambda b, i: (b, 0, i)),
        out_shape=jax.ShapeDtypeStruct((B, C, HW), jnp.float32),
    )(xr, palette)
    return out.reshape(B, C, H, W)


# speedup vs baseline: 2.6826x; 1.0291x over previous
import jax
import jax.numpy as jnp
from jax.experimental import pallas as pl
from jax.experimental.pallas import tpu as pltpu

_TILE = 1792


def _vq_body(x_ref, pal_ref, out_ref):
    xv = x_ref[0]
    T = xv.shape[1]
    pal = pal_ref[...]
    psq = jnp.sum(pal * pal, axis=1, keepdims=True)
    e = jax.lax.dot_general(
        pal.astype(jnp.bfloat16), xv.astype(jnp.bfloat16),
        (((1,), (0,)), ((), ())),
        preferred_element_type=jnp.float32)
    xsq = jnp.sum(xv * xv, axis=0, keepdims=True)
    raw = (xsq + psq) - 2.0 * e
    K = pal.shape[0]
    iota = jax.lax.broadcasted_iota(jnp.int32, (K, T), 0)
    m0 = jnp.maximum(jnp.min(raw, axis=0, keepdims=True), 0.0)
    labels = jnp.min(jnp.where(raw <= m0, iota, K), axis=0)
    onehot = (iota == labels[None, :]).astype(jnp.float32)
    rgb = jax.lax.dot_general(
        pal, onehot, (((0,), (0,)), ((), ())),
        precision=jax.lax.Precision.HIGHEST,
        preferred_element_type=jnp.float32)
    out_ref[0] = rgb


def kernel(x, palette):
    B, C, H, W = x.shape
    HW = H * W
    xr = x.reshape(B, C, HW)
    grid = (B, HW // _TILE)
    out = pl.pallas_call(
        _vq_body,
        grid=grid,
        in_specs=[
            pl.BlockSpec((1, C, _TILE), lambda b, i: (b, 0, i)),
            pl.BlockSpec(palette.shape, lambda b, i: (0, 0)),
        ],
        out_specs=pl.BlockSpec((1, C, _TILE), lambda b, i: (b, 0, i)),
        out_shape=jax.ShapeDtypeStruct((B, C, HW), jnp.float32),
    )(xr, palette)
    return out.reshape(B, C, H, W)


# speedup vs baseline: 5.3948x; 2.0110x over previous
import functools

import jax
import jax.numpy as jnp
from jax import lax
from jax.experimental import pallas as pl
from jax.experimental.pallas import tpu as pltpu
from jax.experimental.pallas import tpu_sc as plsc

_TILE = 1792


def _vq_labels_body(x_ref, pal_ref, lab_ref):
    xv = x_ref[0]
    pal = pal_ref[...]
    psq = jnp.sum(pal * pal, axis=1, keepdims=True)
    e = jax.lax.dot_general(
        pal.astype(jnp.bfloat16), xv.astype(jnp.bfloat16),
        (((1,), (0,)), ((), ())),
        preferred_element_type=jnp.float32)
    xsq = jnp.sum(xv * xv, axis=0, keepdims=True)
    raw = (xsq + psq) - 2.0 * e
    K = pal.shape[0]
    T = xv.shape[1]
    iota = jax.lax.broadcasted_iota(jnp.int32, (K, T), 0)
    m0 = jnp.maximum(jnp.min(raw, axis=0, keepdims=True), 0.0)
    labels = jnp.min(jnp.where(raw <= m0, iota, K), axis=0)
    lab_ref[0] = labels[None, :]


def _labels_tc(xr, palette):
    B, C, HW = xr.shape
    grid = (B, HW // _TILE)
    return pl.pallas_call(
        _vq_labels_body,
        grid=grid,
        in_specs=[
            pl.BlockSpec((1, C, _TILE), lambda b, i: (b, 0, i)),
            pl.BlockSpec(palette.shape, lambda b, i: (0, 0)),
        ],
        out_specs=pl.BlockSpec((1, 1, _TILE), lambda b, i: (b, 0, i)),
        out_shape=jax.ShapeDtypeStruct((B, 1, HW), jnp.int32),
    )(xr, palette)


_NC = 2
_NS = 16
_NW = _NC * _NS


def _sc_gather_body(chunk, hw, lab_hbm, pal_hbm, out_hbm, pal_v, lab_v,
                    out_v0, out_v1, out_v2):
    out_v = (out_v0, out_v1, out_v2)
    wid = lax.axis_index("c") * _NS + lax.axis_index("s")
    wpb = hw // chunk
    b = wid // wpb
    off = (wid % wpb) * chunk
    pix = b * hw + off
    pltpu.sync_copy(pal_hbm, pal_v)
    pltpu.sync_copy(lab_hbm.at[pl.ds(pix, chunk)], lab_v)

    def body(i, carry):
        l16 = lab_v[pl.ds(i * 16, 16)]
        base = l16 * 3
        for ch in range(3):
            out_v[ch][pl.ds(i * 16, 16)] = plsc.load_gather(
                pal_v, [base + ch])
        return carry

    lax.fori_loop(0, chunk // 16, body, 0)
    for ch in range(3):
        pltpu.sync_copy(out_v[ch],
                        out_hbm.at[pl.ds((b * 3 + ch) * hw + off, chunk)])


def _gather_sc(labels, palette, hw):
    n = labels.shape[0]
    chunk = n // _NW
    mesh = plsc.VectorSubcoreMesh(core_axis_name="c", subcore_axis_name="s")
    fn = functools.partial(
        pl.kernel,
        mesh=mesh,
        compiler_params=pltpu.CompilerParams(needs_layout_passes=False),
        out_type=jax.ShapeDtypeStruct((3 * n,), jnp.float32),
        scratch_types=[
            pltpu.VMEM((palette.shape[0] * palette.shape[1],), jnp.float32),
            pltpu.VMEM((chunk,), jnp.int32),
            pltpu.VMEM((chunk,), jnp.float32),
            pltpu.VMEM((chunk,), jnp.float32),
            pltpu.VMEM((chunk,), jnp.float32),
        ],
    )(functools.partial(_sc_gather_body, chunk, hw))
    return fn(labels, palette.reshape(-1))


def kernel(x, palette):
    B, C, H, W = x.shape
    HW = H * W
    xr = x.reshape(B, C, HW)
    labels = _labels_tc(xr, palette).reshape(B * HW)
    out = _gather_sc(labels, palette, HW)
    return out.reshape(B, C, H, W)


# speedup vs baseline: 5.9297x; 1.0991x over previous
import functools

import jax
import jax.numpy as jnp
from jax import lax
from jax.experimental import pallas as pl
from jax.experimental.pallas import tpu as pltpu
from jax.experimental.pallas import tpu_sc as plsc

_TILE = 1792


def _vq_labels_body(x_ref, pal_ref, lab_ref):
    xv = x_ref[0]
    pal = pal_ref[...]
    psq = jnp.sum(pal * pal, axis=1, keepdims=True)
    e2 = jax.lax.dot_general(
        (-2.0 * pal).astype(jnp.bfloat16), xv.astype(jnp.bfloat16),
        (((1,), (0,)), ((), ())),
        preferred_element_type=jnp.float32)
    xsq = jnp.sum(xv * xv, axis=0, keepdims=True)
    raw = (xsq + psq) + e2
    K = pal.shape[0]
    T = xv.shape[1]
    v = jnp.maximum(raw, 0.0)
    ix = jax.lax.broadcasted_iota(jnp.int32, (K, T), 0)
    while v.shape[0] > 1:
        h = v.shape[0] // 2
        va, vb = v[:h], v[h:]
        ia, ib = ix[:h], ix[h:]
        take_b = vb < va
        v = jnp.where(take_b, vb, va)
        ix = jnp.where(take_b, ib, ia)
    lab_ref[0] = ix


def _labels_tc(xr, palette):
    B, C, HW = xr.shape
    grid = (B, HW // _TILE)
    return pl.pallas_call(
        _vq_labels_body,
        grid=grid,
        in_specs=[
            pl.BlockSpec((1, C, _TILE), lambda b, i: (b, 0, i)),
            pl.BlockSpec(palette.shape, lambda b, i: (0, 0)),
        ],
        out_specs=pl.BlockSpec((1, 1, _TILE), lambda b, i: (b, 0, i)),
        out_shape=jax.ShapeDtypeStruct((B, 1, HW), jnp.int32),
    )(xr, palette)


_NC = 2
_NS = 16
_NW = _NC * _NS


def _sc_gather_body(chunk, hw, lab_hbm, pal_hbm, out_hbm, pal_v, lab_v,
                    out_v0, out_v1, out_v2):
    out_v = (out_v0, out_v1, out_v2)
    wid = lax.axis_index("c") * _NS + lax.axis_index("s")
    wpb = hw // chunk
    b = wid // wpb
    off = (wid % wpb) * chunk
    pix = b * hw + off
    pltpu.sync_copy(pal_hbm, pal_v)
    pltpu.sync_copy(lab_hbm.at[pl.ds(pix, chunk)], lab_v)

    def body(i, carry):
        l16 = lab_v[pl.ds(i * 16, 16)]
        base = l16 * 3
        for ch in range(3):
            out_v[ch][pl.ds(i * 16, 16)] = plsc.load_gather(
                pal_v, [base + ch])
        return carry

    lax.fori_loop(0, chunk // 16, body, 0)
    for ch in range(3):
        pltpu.sync_copy(out_v[ch],
                        out_hbm.at[pl.ds((b * 3 + ch) * hw + off, chunk)])


def _gather_sc(labels, palette, hw):
    n = labels.shape[0]
    chunk = n // _NW
    mesh = plsc.VectorSubcoreMesh(core_axis_name="c", subcore_axis_name="s")
    fn = functools.partial(
        pl.kernel,
        mesh=mesh,
        compiler_params=pltpu.CompilerParams(needs_layout_passes=False),
        out_type=jax.ShapeDtypeStruct((3 * n,), jnp.float32),
        scratch_types=[
            pltpu.VMEM((palette.shape[0] * palette.shape[1],), jnp.float32),
            pltpu.VMEM((chunk,), jnp.int32),
            pltpu.VMEM((chunk,), jnp.float32),
            pltpu.VMEM((chunk,), jnp.float32),
            pltpu.VMEM((chunk,), jnp.float32),
        ],
    )(functools.partial(_sc_gather_body, chunk, hw))
    return fn(labels, palette.reshape(-1))


def kernel(x, palette):
    B, C, H, W = x.shape
    HW = H * W
    xr = x.reshape(B, C, HW)
    labels = _labels_tc(xr, palette).reshape(B * HW)
    out = _gather_sc(labels, palette, HW)
    return out.reshape(B, C, H, W)
